# Initial kernel scaffold; baseline (speedup 1.0000x reference)
import jax
import jax.numpy as jnp
from jax.experimental import pallas as pl


def kernel(pretrained, finetuned):
    raise NotImplementedError("write your pallas kernel here")


# speedup vs baseline: 13.4987x; 13.4987x over previous
import math

import jax
import jax.numpy as jnp
from jax import lax
from jax.experimental import pallas as pl
from jax.experimental.pallas import tpu as pltpu
from jax.experimental.pallas import tpu_sc as plsc

N_TOTAL = 16777216
K_SEL = int(0.01 * N_TOTAL)
BIAS = 5.0
SIG_HI = float(1.0 / (1.0 + math.exp(-BIAS)))
SIG_LO = float(1.0 / (1.0 + math.exp(BIAS)))

NC = 2
NS = 16
NW = NC * NS
L = 16

NPT = N_TOTAL // NW
CHUNK = 16384
NCHUNK = NPT // CHUNK
CHUNK5 = 8192
NCHUNK5 = NPT // CHUNK5

CB_ROWS, CB_COLS = 128, 256
FB_ROWS, FB_COLS = 256, 256

_f32 = jnp.float32
_i32 = jnp.int32


def _iota16():
    return lax.iota(_i32, L)


def _extract_lane(x, lane):
    return jnp.sum(jnp.where(_iota16() == lane, x, jnp.zeros_like(x)))


def _worker_id():
    return lax.axis_index("c") * NS + lax.axis_index("s")


def _abs_bits(tv):
    return lax.bitcast_convert_type(tv, _i32) & _i32(0x7FFFFFFF)


def _zero_hist(hist_v, nrows, ncols):
    def zrow(r, c):
        def zcol(t, cc):
            hist_v[r, pl.ds(t * L, L)] = jnp.zeros((L,), _i32)
            return cc
        return lax.fori_loop(0, ncols // L, zcol, c)
    lax.fori_loop(0, nrows, zrow, 0)


def _select_top(h, rowtot_s, nrows, ncols, above0, kk):
    nvpr = ncols // L

    def rowsum(r, c):
        def acc(t, a):
            return a + h[r, pl.ds(t * L, L)]
        a = lax.fori_loop(0, nvpr, acc, jnp.zeros((L,), _i32))
        rowtot_s[r] = jnp.sum(a)
        return c
    lax.fori_loop(0, nrows, rowsum, 0)

    def rowscan(jj, carry):
        above, found, rstar, above_r = carry
        r = nrows - 1 - jj
        t = rowtot_s[r]
        hit = above + t >= kk
        newly = jnp.logical_and(found == 0, hit)
        rstar = jnp.where(newly, r, rstar)
        above_r = jnp.where(newly, above, above_r)
        found = jnp.where(hit, _i32(1), found)
        return above + t, found, rstar, above_r
    _, _, rstar, above_r = lax.fori_loop(
        0, nrows, rowscan, (above0, _i32(0), _i32(0), above0))

    def binscan(jj, carry):
        above, found, bstar, above_b = carry
        vt = nvpr - 1 - jj
        v = h[rstar, pl.ds(vt * L, L)]
        c = plsc.cumsum(v)
        tot = jnp.sum(v)
        g = above + (tot - c + v)
        npos = jnp.max(plsc.all_reduce_population_count(g >= kk))
        newly = jnp.logical_and(found == 0, npos > 0)
        lstar = npos - 1
        cl = _extract_lane(c, lstar)
        bstar = jnp.where(newly, rstar * ncols + vt * L + lstar, bstar)
        above_b = jnp.where(newly, above + tot - cl, above_b)
        found = jnp.where(npos > 0, _i32(1), found)
        return above + tot, found, bstar, above_b
    _, _, bstar, above_b = lax.fori_loop(
        0, nvpr, binscan, (above_r, _i32(0), _i32(0), above_r))
    return bstar, above_b


def _k1_body(p_hbm, f_hbm, tv_hbm, ch_hbm, p_v, f_v, tv_v, hist_v):
    wid = _worker_id()
    _zero_hist(hist_v, CB_ROWS, CB_COLS)
    base0 = wid * NPT

    def chunk(ci, c):
        base = base0 + ci * CHUNK
        pltpu.sync_copy(p_hbm.at[pl.ds(base, CHUNK)], p_v)
        pltpu.sync_copy(f_hbm.at[pl.ds(base, CHUNK)], f_v)

        def inner(j, cc):
            sl = pl.ds(j * L, L)
            tv = f_v[sl] - p_v[sl]
            tv_v[sl] = tv
            bits = _abs_bits(tv)
            coarse = bits >> 16
            cnt, last = plsc.scan_count(coarse)
            plsc.addupdate_scatter(hist_v, [coarse >> 8, coarse & 255], cnt,
                                   mask=last)
            return cc
        lax.fori_loop(0, CHUNK // L, inner, 0)
        pltpu.sync_copy(tv_v, tv_hbm.at[pl.ds(base, CHUNK)])
        return c
    lax.fori_loop(0, NCHUNK, chunk, 0)
    pltpu.sync_copy(hist_v, ch_hbm.at[pl.ds(wid * CB_ROWS, CB_ROWS)])


def _merge_body_for(nrows):
    rpt = nrows // NW

    def body(hall_hbm, out_hbm, acc_v, tmp_v):
        wid = _worker_id()
        r0 = wid * rpt
        pltpu.sync_copy(hall_hbm.at[pl.ds(r0, rpt)], acc_v)

        def rsum(s, c):
            pltpu.sync_copy(hall_hbm.at[pl.ds(s * nrows + r0, rpt)], tmp_v)

            def radd(rr, cc):
                def cadd(u, ccc):
                    sl = pl.ds(u * L, L)
                    acc_v[rr, sl] = acc_v[rr, sl] + tmp_v[rr, sl]
                    return ccc
                return lax.fori_loop(0, 256 // L, cadd, cc)
            return lax.fori_loop(0, rpt, radd, c)
        lax.fori_loop(1, NW, rsum, 0)
        pltpu.sync_copy(acc_v, out_hbm.at[pl.ds(r0, rpt)])
    return body


def _k3_body(tv_hbm, mch_hbm, fh_hbm, tv_v, hist_v, mch_v, rowtot_s):
    wid = _worker_id()
    pltpu.sync_copy(mch_hbm, mch_v)
    bstar, _ = _select_top(mch_v, rowtot_s, CB_ROWS, CB_COLS,
                           _i32(0), _i32(K_SEL))

    _zero_hist(hist_v, FB_ROWS, FB_COLS)
    base0 = wid * NPT

    def chunk(ci, c):
        base = base0 + ci * CHUNK
        pltpu.sync_copy(tv_hbm.at[pl.ds(base, CHUNK)], tv_v)

        def inner(j, cc):
            bits = _abs_bits(tv_v[pl.ds(j * L, L)])
            msk = (bits >> 16) == bstar
            fine = bits & 0xFFFF
            cnt, last = plsc.scan_count(fine, mask=msk)
            plsc.addupdate_scatter(hist_v, [fine >> 8, fine & 255], cnt,
                                   mask=jnp.logical_and(last, msk))
            return cc
        lax.fori_loop(0, CHUNK // L, inner, 0)
        return c
    lax.fori_loop(0, NCHUNK, chunk, 0)
    pltpu.sync_copy(hist_v, fh_hbm.at[pl.ds(wid * FB_ROWS, FB_ROWS)])


def _k5_body(tv_hbm, mch_hbm, mfh_hbm, md_hbm, mk_hbm, prop_hbm,
             tv_v, md_v, mk_v, mch_v, mfh_v, prop_v, rowtot_s):
    wid = _worker_id()
    pltpu.sync_copy(mch_hbm, mch_v)
    pltpu.sync_copy(mfh_hbm, mfh_v)
    bstar, coarse_above = _select_top(mch_v, rowtot_s, CB_ROWS, CB_COLS,
                                      _i32(0), _i32(K_SEL))
    mstar, n_above = _select_top(mfh_v, rowtot_s, FB_ROWS, FB_COLS,
                                 coarse_above, _i32(K_SEL))
    tbits = (bstar << 16) | mstar

    @pl.when(wid == 0)
    def _():
        prop = (_f32(2.0) * n_above.astype(_f32) - _f32(N_TOTAL)) \
            * _f32(BIAS / N_TOTAL)
        prop_v[...] = jnp.where(_iota16() == 0, prop, _f32(0.0))
        pltpu.sync_copy(prop_v, prop_hbm)

    base0 = wid * NPT

    def chunk(ci, c):
        base = base0 + ci * CHUNK5
        pltpu.sync_copy(tv_hbm.at[pl.ds(base, CHUNK5)], tv_v)

        def inner(j, cc):
            sl = pl.ds(j * L, L)
            tv = tv_v[sl]
            m = _abs_bits(tv) > tbits
            mk_v[sl] = jnp.where(m, jnp.full((L,), BIAS, _f32),
                                 jnp.full((L,), -BIAS, _f32))
            md_v[sl] = tv * jnp.where(m, jnp.full((L,), SIG_HI, _f32),
                                      jnp.full((L,), SIG_LO, _f32))
            return cc
        lax.fori_loop(0, CHUNK5 // L, inner, 0)
        pltpu.sync_copy(md_v, md_hbm.at[pl.ds(base, CHUNK5)])
        pltpu.sync_copy(mk_v, mk_hbm.at[pl.ds(base, CHUNK5)])
        return c
    lax.fori_loop(0, NCHUNK5, chunk, 0)


def _mesh():
    return plsc.VectorSubcoreMesh(core_axis_name="c", subcore_axis_name="s",
                                  num_cores=NC, num_subcores=NS)


_CPARAMS = pltpu.CompilerParams(needs_layout_passes=False)


def kernel(pretrained, finetuned):
    sds = jax.ShapeDtypeStruct
    k1 = pl.kernel(
        _k1_body,
        out_type=(sds((N_TOTAL,), _f32),
                  sds((NW * CB_ROWS, CB_COLS), _i32)),
        mesh=_mesh(),
        compiler_params=_CPARAMS,
        scratch_types=[
            pltpu.VMEM((CHUNK,), _f32),
            pltpu.VMEM((CHUNK,), _f32),
            pltpu.VMEM((CHUNK,), _f32),
            pltpu.VMEM((CB_ROWS, CB_COLS), _i32),
        ],
    )
    tv, ch_all = k1(pretrained, finetuned)

    k2 = pl.kernel(
        _merge_body_for(CB_ROWS),
        out_type=sds((CB_ROWS, CB_COLS), _i32),
        mesh=_mesh(),
        compiler_params=_CPARAMS,
        scratch_types=[
            pltpu.VMEM((CB_ROWS // NW, CB_COLS), _i32),
            pltpu.VMEM((CB_ROWS // NW, CB_COLS), _i32),
        ],
    )
    mch = k2(ch_all)

    k3 = pl.kernel(
        _k3_body,
        out_type=sds((NW * FB_ROWS, FB_COLS), _i32),
        mesh=_mesh(),
        compiler_params=_CPARAMS,
        scratch_types=[
            pltpu.VMEM((CHUNK,), _f32),
            pltpu.VMEM((FB_ROWS, FB_COLS), _i32),
            pltpu.VMEM((CB_ROWS, CB_COLS), _i32),
            pltpu.SMEM((FB_ROWS,), _i32),
        ],
    )
    fh_all = k3(tv, mch)

    k4 = pl.kernel(
        _merge_body_for(FB_ROWS),
        out_type=sds((FB_ROWS, FB_COLS), _i32),
        mesh=_mesh(),
        compiler_params=_CPARAMS,
        scratch_types=[
            pltpu.VMEM((FB_ROWS // NW, FB_COLS), _i32),
            pltpu.VMEM((FB_ROWS // NW, FB_COLS), _i32),
        ],
    )
    mfh = k4(fh_all)

    k5 = pl.kernel(
        _k5_body,
        out_type=(sds((N_TOTAL,), _f32), sds((N_TOTAL,), _f32),
                  sds((L,), _f32)),
        mesh=_mesh(),
        compiler_params=_CPARAMS,
        scratch_types=[
            pltpu.VMEM((CHUNK5,), _f32),
            pltpu.VMEM((CHUNK5,), _f32),
            pltpu.VMEM((CHUNK5,), _f32),
            pltpu.VMEM((CB_ROWS, CB_COLS), _i32),
            pltpu.VMEM((FB_ROWS, FB_COLS), _i32),
            pltpu.VMEM((L,), _f32),
            pltpu.SMEM((FB_ROWS,), _i32),
        ],
    )
    masked_delta, mask, propv = k5(tv, mch, mfh)

    return (masked_delta, mask, propv[0])
